# Initial kernel scaffold; baseline (speedup 1.0000x reference)
import jax
import jax.numpy as jnp
from jax.experimental import pallas as pl


def kernel(xyz_q, lat_rep, xyz, points, Wq, Wk, Wv, Wkg, Wvg, gW1, gb1, gW2, gb2, peW, peb, ieW, ieb, fccW, fccb, b0W, b0b, b1W, b1b, foW, fob):
    raise NotImplementedError("write your pallas kernel here")


# speedup vs baseline: 5.4523x; 5.4523x over previous
import functools

import jax
import jax.numpy as jnp
from jax import lax
from jax.experimental import pallas as pl
from jax.experimental.pallas import tpu as pltpu
from jax.experimental.pallas import tpu_sc as plsc

B = 2
NQ = 2048
N = 2048
LAT = 256
DIM = 256
NN = 16
HID = 128
NB = 5
OUT = 3

VX = 384
QA = 256
QC = 128
FREQS = (1.0, 8.75, 16.5, 24.25, 32.0)

F32 = jnp.float32


def _topk_body(xq_ref, xt_ref, o_ref):
    b = pl.program_id(0)
    xq = xq_ref[0]
    xt = xt_ref[0]
    qx = xq[:, 0:1]
    qy = xq[:, 1:2]
    px = xt[0:1, :]
    py = xt[1:2, :]
    d = (qx - px) ** 2 + (qy - py) ** 2
    it = lax.broadcasted_iota(jnp.int32, (QA, N), 1).astype(F32)
    cols = []
    for _ in range(NN):
        v = jnp.min(d, axis=1, keepdims=True)
        cand = jnp.where(d == v, it, F32(N))
        i = jnp.min(cand, axis=1, keepdims=True)
        d = jnp.where(it == i, F32(3e38), d)
        cols.append(i)
    idx = jnp.concatenate(cols, axis=1).astype(jnp.int32) + b * N
    o_ref[0] = idx


def _topk(xyz_q, xyz_t):
    return pl.pallas_call(
        _topk_body,
        grid=(B, NQ // QA),
        in_specs=[
            pl.BlockSpec((1, QA, 2), lambda b, q: (b, q, 0)),
            pl.BlockSpec((1, 2, N), lambda b, q: (b, 0, 0)),
        ],
        out_specs=pl.BlockSpec((1, QA, NN), lambda b, q: (b, q, 0)),
        out_shape=jax.ShapeDtypeStruct((B, NQ, NN), jnp.int32),
    )(xyz_q, xyz_t)


def _proj_body(pts_ref, xyz_ref, lat_ref, wq_ref, wk_ref, wv_ref, wkg_ref,
               wvg_ref, gw1_ref, gb1_ref, gw2_ref, pew_ref, peb_ref,
               kg_ref, v_ref, wpe2_ref, arow_ref, lg_ref, vg_ref):
    dot = functools.partial(jnp.dot, preferred_element_type=F32)
    pts = pts_ref[0]
    gw1 = gw1_ref[...]
    wk1 = dot(wk_ref[...], gw1)
    kg_ref[0] = dot(pts, wk1)
    v_ref[0] = jnp.concatenate(
        [dot(pts, wv_ref[...]), xyz_ref[0],
         jnp.zeros((N, VX - DIM - 3), F32)], axis=1)
    pewg = dot(pew_ref[...], gw1)
    wpe2_ref[...] = jnp.concatenate([pewg, pew_ref[...]], axis=1)
    lat = lat_ref[...]
    qg = dot(lat, wq_ref[...])
    arow_ref[:, 0, :] = dot(qg, gw1) + dot(peb_ref[...], gw1) + gb1_ref[...]
    kgl = dot(lat, wkg_ref[...])
    hg = jnp.maximum(dot(qg - kgl, gw1) + gb1_ref[...], 0.0)
    lg_ref[:, 0, :] = dot(hg, gw2_ref[...])
    vg_ref[:, 0, :] = dot(lat, wvg_ref[...])


def _proj(points, xyz, lat_rep, Wq, Wk, Wv, Wkg, Wvg, gW1, gb1r, gW2, peW,
          pebr):
    full2 = lambda a: pl.BlockSpec(a.shape, lambda b: (0,) * a.ndim)
    return pl.pallas_call(
        _proj_body,
        grid=(B,),
        in_specs=[
            pl.BlockSpec((1, N, LAT), lambda b: (b, 0, 0)),
            pl.BlockSpec((1, N, 3), lambda b: (b, 0, 0)),
            full2(lat_rep), full2(Wq), full2(Wk), full2(Wv), full2(Wkg),
            full2(Wvg), full2(gW1), full2(gb1r), full2(gW2), full2(peW),
            full2(pebr),
        ],
        out_specs=[
            pl.BlockSpec((1, N, DIM), lambda b: (b, 0, 0)),
            pl.BlockSpec((1, N, VX), lambda b: (b, 0, 0)),
            pl.BlockSpec((33, 2 * DIM), lambda b: (0, 0)),
            pl.BlockSpec((B, 1, DIM), lambda b: (0, 0, 0)),
            pl.BlockSpec((B, 1, DIM), lambda b: (0, 0, 0)),
            pl.BlockSpec((B, 1, DIM), lambda b: (0, 0, 0)),
        ],
        out_shape=[
            jax.ShapeDtypeStruct((B, N, DIM), F32),
            jax.ShapeDtypeStruct((B, N, VX), F32),
            jax.ShapeDtypeStruct((33, 2 * DIM), F32),
            jax.ShapeDtypeStruct((B, 1, DIM), F32),
            jax.ShapeDtypeStruct((B, 1, DIM), F32),
            jax.ShapeDtypeStruct((B, 1, DIM), F32),
        ],
    )(points, xyz, lat_rep, Wq, Wk, Wv, Wkg, Wvg, gW1, gb1r, gW2, peW, pebr)


def _sc_gather(idx_flat, kgf, vxf):
    tot = B * NQ * NN
    nc, ns = 2, 16
    nw = nc * ns
    per_w = tot // nw
    ch = 128
    n_ch = per_w // ch
    mesh = plsc.VectorSubcoreMesh(core_axis_name="c", subcore_axis_name="s")

    @functools.partial(
        pl.kernel,
        mesh=mesh,
        out_type=[
            jax.ShapeDtypeStruct((tot, DIM), F32),
            jax.ShapeDtypeStruct((tot, VX), F32),
        ],
        scratch_types=[
            pltpu.VMEM((ch,), jnp.int32),
            pltpu.VMEM((ch, DIM), F32),
            pltpu.VMEM((ch, VX), F32),
            pltpu.SemaphoreType.DMA,
        ],
    )
    def gk(idx_hbm, kg_hbm, vx_hbm, okg_hbm, ovx_hbm, idx_v, kg_v, vx_v, sem):
        wid = lax.axis_index("s") * nc + lax.axis_index("c")
        base = wid * per_w

        @pl.loop(0, n_ch)
        def _(c):
            off = base + c * ch
            pltpu.sync_copy(idx_hbm.at[pl.ds(off, ch)], idx_v)
            pltpu.async_copy(kg_hbm.at[idx_v], kg_v, sem).wait()
            pltpu.sync_copy(kg_v, okg_hbm.at[pl.ds(off, ch)])
            pltpu.async_copy(vx_hbm.at[idx_v], vx_v, sem).wait()
            pltpu.sync_copy(vx_v, ovx_hbm.at[pl.ds(off, ch)])

    return gk(idx_flat, kgf, vxf)


def _attn_body(kgg_ref, vxg_ref, xqr_ref, arow_ref, lg_ref, vg_ref,
               wpe2_ref, gw2_ref, peb_ref, iew_ref, ieb_ref, fccw_ref,
               fccb_ref, b0w_ref, b0b_ref, b1w_ref, b1b_ref, fow_ref,
               fob_ref, o_ref):
    dot = functools.partial(jnp.dot, preferred_element_type=F32)
    vxg = vxg_ref[...]
    xq = xqr_ref[...]
    pd = jnp.concatenate(
        [xq[:, 0:1] - vxg[:, DIM:DIM + 1], xq[:, 1:2] - vxg[:, DIM + 1:DIM + 2],
         vxg[:, DIM + 2:DIM + 3]],
        axis=1)
    pieces = [pd]
    for f in FREQS:
        sc = pd * F32(f)
        pieces.append(jnp.sin(sc))
        pieces.append(jnp.cos(sc))
    pe33 = jnp.concatenate(pieces, axis=1)
    pe2 = dot(pe33, wpe2_ref[...])
    hidden = jnp.maximum(arow_ref[0] + pe2[:, :DIM] - kgg_ref[...], 0.0)
    logits = dot(hidden, gw2_ref[...])
    veff = vxg[:, :DIM] + pe2[:, DIM:] + peb_ref[...]
    l3 = logits.reshape(QC, NN, DIM)
    v3 = veff.reshape(QC, NN, DIM)
    lg = lg_ref[0]
    m = jnp.maximum(jnp.max(l3, axis=1), lg)
    e3 = jnp.exp(l3 - m[:, None, :])
    eg = jnp.exp(lg - m)
    s = jnp.sum(e3 * v3, axis=1)
    zden = jnp.sum(e3, axis=1) + eg
    y = (s + eg * vg_ref[0]) / zden
    net = dot(y, iew_ref[...]) + ieb_ref[...]
    for i in range(NB):
        net = net + dot(y, fccw_ref[i]) + fccb_ref[i:i + 1]
        h = dot(jnp.maximum(net, 0.0), b0w_ref[i]) + b0b_ref[i:i + 1]
        dx = dot(jnp.maximum(h, 0.0), b1w_ref[i]) + b1b_ref[i:i + 1]
        net = net + dx
    o_ref[...] = dot(jnp.maximum(net, 0.0), fow_ref[...]) + fob_ref[...]


def _attn(kgg, vxg, xqr, arow, lgl, vgl, wpe2, gW2, pebr, ieW, iebr,
          fccW, fccb, b0W, b0b, b1W, b1b, foW, fobr):
    R = QC * NN
    G = B * NQ // QC
    per_b = NQ // QC
    full = lambda a: pl.BlockSpec(a.shape, lambda i: (0,) * a.ndim)
    glob = pl.BlockSpec((1, 1, DIM), lambda i: (i // per_b, 0, 0))
    return pl.pallas_call(
        _attn_body,
        grid=(G,),
        in_specs=[
            pl.BlockSpec((R, DIM), lambda i: (i, 0)),
            pl.BlockSpec((R, VX), lambda i: (i, 0)),
            pl.BlockSpec((R, 2), lambda i: (i, 0)),
            glob, glob, glob,
            full(wpe2), full(gW2), full(pebr), full(ieW), full(iebr),
            full(fccW), full(fccb), full(b0W), full(b0b), full(b1W),
            full(b1b), full(foW), full(fobr),
        ],
        out_specs=pl.BlockSpec((QC, OUT), lambda i: (i, 0)),
        out_shape=jax.ShapeDtypeStruct((B * NQ, OUT), F32),
    )(kgg, vxg, xqr, arow, lgl, vgl, wpe2, gW2, pebr, ieW, iebr,
      fccW, fccb, b0W, b0b, b1W, b1b, foW, fobr)


def kernel(xyz_q, lat_rep, xyz, points, Wq, Wk, Wv, Wkg, Wvg, gW1, gb1,
           gW2, gb2, peW, peb, ieW, ieb, fccW, fccb, b0W, b0b, b1W, b1b,
           foW, fob):
    del gb2
    gb1r = gb1.reshape(1, DIM)
    pebr = peb.reshape(1, DIM)
    iebr = ieb.reshape(1, HID)
    fobr = fob.reshape(1, OUT)

    xyz_t = jnp.transpose(xyz[:, :, :2], (0, 2, 1))
    idx = _topk(xyz_q, xyz_t)
    kg, vx, wpe2, arow, lgl, vgl = _proj(
        points, xyz, lat_rep, Wq, Wk, Wv, Wkg, Wvg, gW1, gb1r, gW2, peW,
        pebr)

    idx_flat = idx.reshape(B * NQ * NN)
    kgf = kg.reshape(B * N, DIM)
    vxf = vx.reshape(B * N, VX)
    kgg, vxg = _sc_gather(idx_flat, kgf, vxf)

    xqr = jnp.repeat(xyz_q.reshape(B * NQ, 2), NN, axis=0)
    out = _attn(kgg, vxg, xqr, arow, lgl, vgl, wpe2, gW2, pebr, ieW,
                iebr, fccW, fccb, b0W, b0b, b1W, b1b, foW, fobr)
    return out.reshape(B, NQ, OUT)
